# baseline (speedup 1.0000x reference)
import jax
import jax.numpy as jnp
from jax.experimental import pallas as pl
from jax.experimental.pallas import tpu as pltpu

DIM = 4096
INTER = 14336
BLK = 256
NSTEP = INTER // BLK


def _ffn_kernel(x_ref, w1_ref, w2_ref, o_ref):
    @pl.when(pl.program_id(0) == 0)
    def _init():
        o_ref[...] = jnp.zeros_like(o_ref)

    h = jax.lax.dot_general(
        x_ref[...], w1_ref[...],
        dimension_numbers=(((1,), (1,)), ((), ())),
        preferred_element_type=jnp.float32,
    )
    h = jnp.maximum(h, 0.0)
    o_ref[...] += jax.lax.dot_general(
        h, w2_ref[...],
        dimension_numbers=(((1,), (1,)), ((), ())),
        preferred_element_type=jnp.float32,
    )


@jax.jit
def kernel(x, W1, W2):
    b, t, d = x.shape
    xt = x.reshape(b * t, d)
    out = pl.pallas_call(
        _ffn_kernel,
        grid=(NSTEP,),
        in_specs=[
            pl.BlockSpec((b * t, DIM), lambda i: (0, 0)),
            pl.BlockSpec((BLK, DIM), lambda i: (i, 0)),
            pl.BlockSpec((DIM, BLK), lambda i: (0, i)),
        ],
        out_specs=pl.BlockSpec((b * t, DIM), lambda i: (0, 0)),
        out_shape=jax.ShapeDtypeStruct((b * t, DIM), jnp.float32),
        compiler_params=pltpu.CompilerParams(vmem_limit_bytes=128 * 1024 * 1024),
    )(xt, W1, W2)
    return out.reshape(b, t, d)


# speedup vs baseline: 1.1139x; 1.1139x over previous
import jax
import jax.numpy as jnp
from jax.experimental import pallas as pl
from jax.experimental.pallas import tpu as pltpu

DIM = 4096
INTER = 14336
T = 32
BLK = 512
N1 = INTER // BLK
DBLK = 128
N2 = DIM // DBLK


def _ffn_kernel(xt_ref, w1_ref, w2_ref, o_ref, h_ref):
    i = pl.program_id(0)

    @pl.when(i < N1)
    def _phase1():
        h = jax.lax.dot_general(
            w1_ref[...], xt_ref[...],
            dimension_numbers=(((1,), (0,)), ((), ())),
            preferred_element_type=jnp.float32,
        )
        h_ref[pl.ds(i * BLK, BLK), :] = jnp.maximum(h, 0.0)

    @pl.when(i >= N1)
    def _phase2():
        o_ref[...] = jax.lax.dot_general(
            w2_ref[...], h_ref[...],
            dimension_numbers=(((1,), (0,)), ((), ())),
            preferred_element_type=jnp.float32,
        )


@jax.jit
def kernel(x, W1, W2):
    b, t, d = x.shape
    xt = x.reshape(b * t, d).T
    out_t = pl.pallas_call(
        _ffn_kernel,
        grid=(N1 + N2,),
        in_specs=[
            pl.BlockSpec((DIM, T), lambda i: (0, 0)),
            pl.BlockSpec((BLK, DIM), lambda i: (jnp.minimum(i, N1 - 1), 0)),
            pl.BlockSpec((DBLK, INTER), lambda i: (jnp.maximum(i - N1, 0), 0)),
        ],
        out_specs=pl.BlockSpec((DBLK, T), lambda i: (jnp.maximum(i - N1, 0), 0)),
        out_shape=jax.ShapeDtypeStruct((DIM, T), jnp.float32),
        scratch_shapes=[pltpu.MemorySpace.VMEM((INTER, T), jnp.float32)],
    )(xt, W1, W2)
    return out_t.T.reshape(b, t, d)


# speedup vs baseline: 1.1244x; 1.0094x over previous
import jax
import jax.numpy as jnp
from jax.experimental import pallas as pl
from jax.experimental.pallas import tpu as pltpu

DIM = 4096
INTER = 14336
T = 32

C1 = 128
NC1 = INTER // C1
NB1 = 12
C2 = 128
NC2 = DIM // C2
NB2 = 4


def _ffn_kernel(xt_ref, w1_hbm, w2_hbm, o_ref, w1buf, w2buf, h_ref, sem1, sem2):
    def start1(c):
        b = jax.lax.rem(c, NB1)
        pltpu.make_async_copy(
            w1_hbm.at[pl.ds(c * C1, C1), :], w1buf.at[b], sem1.at[b]).start()

    def start2(c):
        b = jax.lax.rem(c, NB2)
        pltpu.make_async_copy(
            w2_hbm.at[pl.ds(c * C2, C2), :], w2buf.at[b], sem2.at[b]).start()

    for c in range(NB1):
        start1(c)

    def body1(c, carry):
        b = jax.lax.rem(c, NB1)
        pltpu.make_async_copy(
            w1_hbm.at[pl.ds(c * C1, C1), :], w1buf.at[b], sem1.at[b]).wait()
        h = jax.lax.dot_general(
            w1buf[b], xt_ref[...],
            dimension_numbers=(((1,), (0,)), ((), ())),
            preferred_element_type=jnp.float32,
        )
        h_ref[pl.ds(c * C1, C1), :] = jnp.maximum(h, 0.0)

        @pl.when(c + NB1 < NC1)
        def _():
            start1(c + NB1)

        @pl.when((c + NB1 >= NC1) & (c + NB1 < NC1 + NB2))
        def _():
            start2(c + NB1 - NC1)

        return carry

    jax.lax.fori_loop(0, NC1, body1, 0)

    def body2(c, carry):
        b = jax.lax.rem(c, NB2)
        pltpu.make_async_copy(
            w2_hbm.at[pl.ds(c * C2, C2), :], w2buf.at[b], sem2.at[b]).wait()
        o_ref[pl.ds(c * C2, C2), :] = jax.lax.dot_general(
            w2buf[b], h_ref[...],
            dimension_numbers=(((1,), (0,)), ((), ())),
            preferred_element_type=jnp.float32,
        )

        @pl.when(c + NB2 < NC2)
        def _():
            start2(c + NB2)

        return carry

    jax.lax.fori_loop(0, NC2, body2, 0)


@jax.jit
def kernel(x, W1, W2):
    b, t, d = x.shape
    xt = x.reshape(b * t, d).T
    out_t = pl.pallas_call(
        _ffn_kernel,
        in_specs=[
            pl.BlockSpec(memory_space=pltpu.MemorySpace.VMEM),
            pl.BlockSpec(memory_space=pltpu.MemorySpace.HBM),
            pl.BlockSpec(memory_space=pltpu.MemorySpace.HBM),
        ],
        out_specs=pl.BlockSpec(memory_space=pltpu.MemorySpace.VMEM),
        out_shape=jax.ShapeDtypeStruct((DIM, T), jnp.float32),
        compiler_params=pltpu.CompilerParams(vmem_limit_bytes=100 * 1024 * 1024),
        scratch_shapes=[
            pltpu.MemorySpace.VMEM((NB1, C1, DIM), jnp.float32),
            pltpu.MemorySpace.VMEM((NB2, C2, INTER), jnp.float32),
            pltpu.MemorySpace.VMEM((INTER, T), jnp.float32),
            pltpu.SemaphoreType.DMA((NB1,)),
            pltpu.SemaphoreType.DMA((NB2,)),
        ],
    )(xt, W1, W2)
    return out_t.T.reshape(b, t, d)
